# Initial kernel scaffold; baseline (speedup 1.0000x reference)
import functools

import jax
import jax.numpy as jnp
from jax import lax
from jax.experimental import pallas as pl
from jax.experimental.pallas import tpu as pltpu
from jax.experimental.pallas import tpu_sc as plsc

CHAR_VOCAB = 1000
WORD_VOCAB = 1000000
OUT_DIM = 64
B, L = 4096, 200
N = B * L

_NC, _NS = 2, 16
NW = _NC * _NS
PER_W = N // NW
CHUNK = 128
N_CHUNKS = PER_W // CHUNK

_PROJ_BLOCK = 8000


def _proj_body(tab_ref, w_ref, out_ref):
    out_ref[...] = jnp.dot(tab_ref[...], w_ref[...],
                           preferred_element_type=jnp.float32)


def _project_table(word_table, W_dense):
    n_blocks = WORD_VOCAB // _PROJ_BLOCK
    return pl.pallas_call(
        _proj_body,
        grid=(n_blocks,),
        in_specs=[
            pl.BlockSpec((_PROJ_BLOCK, OUT_DIM), lambda i: (i, 0)),
            pl.BlockSpec((OUT_DIM, OUT_DIM), lambda i: (0, 0)),
        ],
        out_specs=pl.BlockSpec((_PROJ_BLOCK, OUT_DIM), lambda i: (i, 0)),
        out_shape=jax.ShapeDtypeStruct((WORD_VOCAB, OUT_DIM), jnp.float32),
    )(word_table, W_dense)


def _sc_body(proj_hbm, char_hbm, widx_hbm, cidx_hbm, out_hbm,
             widx_v, cidx_v, rows_v, sem):
    wid = lax.axis_index("s") * _NC + lax.axis_index("c")
    row0 = wid * N_CHUNKS
    base = wid * PER_W

    pltpu.sync_copy(widx_hbm.at[pl.ds(row0, N_CHUNKS)], widx_v)
    pltpu.sync_copy(cidx_hbm.at[pl.ds(row0, N_CHUNKS)], cidx_v)

    def step(j, carry):
        pltpu.sync_copy(proj_hbm.at[widx_v.at[j]], rows_v)
        pltpu.sync_copy(char_hbm.at[cidx_v.at[j]], rows_v, add=True)
        pltpu.sync_copy(rows_v, out_hbm.at[pl.ds(base + j * CHUNK, CHUNK)])
        return carry

    lax.fori_loop(0, N_CHUNKS, step, 0)


def _sc_mix(proj, char_table, widx, cidx):
    mesh = plsc.VectorSubcoreMesh(core_axis_name="c", subcore_axis_name="s")
    return pl.kernel(
        _sc_body,
        out_type=jax.ShapeDtypeStruct((N, OUT_DIM), jnp.float32),
        mesh=mesh,
        scratch_types=[
            pltpu.VMEM((N_CHUNKS, CHUNK), jnp.int32),
            pltpu.VMEM((N_CHUNKS, CHUNK), jnp.int32),
            pltpu.VMEM((CHUNK, OUT_DIM), jnp.float32),
            pltpu.SemaphoreType.DMA,
        ],
    )(proj, char_table, widx, cidx)


def kernel(char_id, word_id, char_table, word_table, W_dense):
    proj = _project_table(word_table, W_dense)
    widx = word_id.reshape(NW * N_CHUNKS, CHUNK).astype(jnp.int32)
    cidx = char_id.reshape(NW * N_CHUNKS, CHUNK).astype(jnp.int32)
    out = _sc_mix(proj, char_table, widx, cidx)
    return out.reshape(B, L, OUT_DIM)


# speedup vs baseline: 1.6406x; 1.6406x over previous
import functools

import jax
import jax.numpy as jnp
from jax import lax
from jax.experimental import pallas as pl
from jax.experimental.pallas import tpu as pltpu
from jax.experimental.pallas import tpu_sc as plsc

CHAR_VOCAB = 1000
WORD_VOCAB = 1000000
OUT_DIM = 64
B, L = 4096, 200
N = B * L

_NC, _NS = 2, 16
NW = _NC * _NS
PER_W = N // NW
CHUNK = 128
N_CHUNKS = PER_W // CHUNK

_PROJ_BLOCK = 8000


def _proj_body(tab_ref, w_ref, out_ref):
    out_ref[...] = jnp.dot(tab_ref[...], w_ref[...],
                           preferred_element_type=jnp.float32)


def _project_table(word_table, W_dense):
    n_blocks = WORD_VOCAB // _PROJ_BLOCK
    return pl.pallas_call(
        _proj_body,
        grid=(n_blocks,),
        in_specs=[
            pl.BlockSpec((_PROJ_BLOCK, OUT_DIM), lambda i: (i, 0)),
            pl.BlockSpec((OUT_DIM, OUT_DIM), lambda i: (0, 0)),
        ],
        out_specs=pl.BlockSpec((_PROJ_BLOCK, OUT_DIM), lambda i: (i, 0)),
        out_shape=jax.ShapeDtypeStruct((WORD_VOCAB, OUT_DIM), jnp.float32),
    )(word_table, W_dense)


def _sc_body(proj_hbm, char_hbm, widx_hbm, cidx_hbm, out_hbm,
             widx_v, cidx_v, rows_v, sem):
    wid = lax.axis_index("s") * _NC + lax.axis_index("c")
    row0 = wid * N_CHUNKS
    base = wid * PER_W

    pltpu.sync_copy(widx_hbm.at[pl.ds(row0, N_CHUNKS)], widx_v)
    pltpu.sync_copy(cidx_hbm.at[pl.ds(row0, N_CHUNKS)], cidx_v)

    def step(j, carry):
        pltpu.sync_copy(proj_hbm.at[widx_v.at[j]], rows_v)
        pltpu.sync_copy(char_hbm.at[cidx_v.at[j]], rows_v, add=True)
        pltpu.sync_copy(rows_v, out_hbm.at[pl.ds(base + j * CHUNK, CHUNK)])
        return carry

    lax.fori_loop(0, N_CHUNKS, step, 0)


def _sc_mix(proj, char_table, widx, cidx):
    mesh = plsc.VectorSubcoreMesh(core_axis_name="c", subcore_axis_name="s")
    return pl.kernel(
        _sc_body,
        out_type=jax.ShapeDtypeStruct((N, OUT_DIM), jnp.float32),
        mesh=mesh,
        scratch_types=[
            pltpu.VMEM((N_CHUNKS, CHUNK), jnp.int32),
            pltpu.VMEM((N_CHUNKS, CHUNK), jnp.int32),
            pltpu.VMEM((CHUNK, OUT_DIM), jnp.float32),
            pltpu.SemaphoreType.DMA,
        ],
        compiler_params=pltpu.CompilerParams(use_tc_tiling_on_sc=False),
    )(proj, char_table, widx, cidx)


def kernel(char_id, word_id, char_table, word_table, W_dense):
    proj = _project_table(word_table, W_dense)
    widx = word_id.reshape(NW * N_CHUNKS, CHUNK).astype(jnp.int32)
    cidx = char_id.reshape(NW * N_CHUNKS, CHUNK).astype(jnp.int32)
    out = _sc_mix(proj, char_table, widx, cidx)
    return out.reshape(B, L, OUT_DIM)


# speedup vs baseline: 1.7807x; 1.0854x over previous
import functools

import jax
import jax.numpy as jnp
from jax import lax
from jax.experimental import pallas as pl
from jax.experimental.pallas import tpu as pltpu
from jax.experimental.pallas import tpu_sc as plsc

CHAR_VOCAB = 1000
WORD_VOCAB = 1000000
OUT_DIM = 64
B, L = 4096, 200
N = B * L

_NC, _NS = 2, 16
NW = _NC * _NS
PER_W = N // NW
CHUNK = 128
N_CHUNKS = PER_W // CHUNK

_PROJ_BLOCK = 8000


def _proj_body(tab_ref, w_ref, out_ref):
    out_ref[...] = jnp.dot(tab_ref[...], w_ref[...],
                           preferred_element_type=jnp.float32)


def _project_table(word_table, W_dense):
    n_blocks = WORD_VOCAB // _PROJ_BLOCK
    return pl.pallas_call(
        _proj_body,
        grid=(n_blocks,),
        in_specs=[
            pl.BlockSpec((_PROJ_BLOCK, OUT_DIM), lambda i: (i, 0)),
            pl.BlockSpec((OUT_DIM, OUT_DIM), lambda i: (0, 0)),
        ],
        out_specs=pl.BlockSpec((_PROJ_BLOCK, OUT_DIM), lambda i: (i, 0)),
        out_shape=jax.ShapeDtypeStruct((WORD_VOCAB, OUT_DIM), jnp.float32),
    )(word_table, W_dense)


_NBUF = 4
_NGROUPS = N_CHUNKS // _NBUF


def _sc_body(proj_hbm, char_hbm, widx_hbm, cidx_hbm, out_hbm,
             widx_v, cidx_v, rows_v,
             semw0, semw1, semw2, semw3,
             semc0, semc1, semc2, semc3,
             semo0, semo1, semo2, semo3):
    semw = [semw0, semw1, semw2, semw3]
    semc = [semc0, semc1, semc2, semc3]
    semo = [semo0, semo1, semo2, semo3]
    wid = lax.axis_index("s") * _NC + lax.axis_index("c")
    row0 = wid * N_CHUNKS
    base = wid * PER_W

    pltpu.sync_copy(widx_hbm.at[pl.ds(row0, N_CHUNKS)], widx_v)
    pltpu.sync_copy(cidx_hbm.at[pl.ds(row0, N_CHUNKS)], cidx_v)

    def group(g, carry):
        j0 = g * _NBUF

        @pl.when(g > 0)
        def _():
            for p in range(_NBUF):
                pltpu.make_async_copy(
                    rows_v.at[p], out_hbm.at[pl.ds(base, CHUNK)],
                    semo[p]).wait()

        gw = []
        for p in range(_NBUF):
            gw.append(pltpu.async_copy(
                proj_hbm.at[widx_v.at[j0 + p]], rows_v.at[p], semw[p]))
        ga = []
        for p in range(_NBUF):
            gw[p].wait()
            ga.append(pltpu.async_copy(
                char_hbm.at[cidx_v.at[j0 + p]], rows_v.at[p], semc[p],
                add=True))
        for p in range(_NBUF):
            ga[p].wait()
            pltpu.async_copy(
                rows_v.at[p],
                out_hbm.at[pl.ds(base + (j0 + p) * CHUNK, CHUNK)], semo[p])
        return carry

    lax.fori_loop(0, _NGROUPS, group, 0)

    for p in range(_NBUF):
        pltpu.make_async_copy(
            rows_v.at[p], out_hbm.at[pl.ds(base, CHUNK)], semo[p]).wait()


def _sc_mix(proj, char_table, widx, cidx):
    mesh = plsc.VectorSubcoreMesh(core_axis_name="c", subcore_axis_name="s")
    return pl.kernel(
        _sc_body,
        out_type=jax.ShapeDtypeStruct((N, OUT_DIM), jnp.float32),
        mesh=mesh,
        scratch_types=[
            pltpu.VMEM((N_CHUNKS, CHUNK), jnp.int32),
            pltpu.VMEM((N_CHUNKS, CHUNK), jnp.int32),
            pltpu.VMEM((_NBUF, CHUNK, OUT_DIM), jnp.float32),
        ] + [pltpu.SemaphoreType.DMA] * (3 * _NBUF),
        compiler_params=pltpu.CompilerParams(use_tc_tiling_on_sc=False),
    )(proj, char_table, widx, cidx)


def kernel(char_id, word_id, char_table, word_table, W_dense):
    proj = _project_table(word_table, W_dense)
    widx = word_id.reshape(NW * N_CHUNKS, CHUNK).astype(jnp.int32)
    cidx = char_id.reshape(NW * N_CHUNKS, CHUNK).astype(jnp.int32)
    out = _sc_mix(proj, char_table, widx, cidx)
    return out.reshape(B, L, OUT_DIM)
